# baseline (speedup 1.0000x reference)
import functools

import jax
import jax.numpy as jnp
from jax import lax
from jax.experimental import pallas as pl
from jax.experimental.pallas import tpu as pltpu
from jax.experimental.pallas import tpu_sc as plsc

B = 16384
D = 64

NC = 2
NS = 16
NW = NC * NS
B_PER_W = B // NW
CHUNK = 128
NCHUNK = B_PER_W // CHUNK


@functools.lru_cache(maxsize=None)
def _build_gather():
    mesh = plsc.VectorSubcoreMesh(core_axis_name="c", subcore_axis_name="s",
                                  num_cores=NC)

    @functools.partial(
        pl.kernel,
        mesh=mesh,
        out_type=(
            jax.ShapeDtypeStruct((B, D), jnp.float32),
            jax.ShapeDtypeStruct((B, D), jnp.float32),
        ),
        scratch_types=[
            pltpu.VMEM((NCHUNK, CHUNK), jnp.int32),
            pltpu.VMEM((B_PER_W, D), jnp.float32),
            pltpu.VMEM((NCHUNK, CHUNK), jnp.int32),
            pltpu.VMEM((B_PER_W, D), jnp.float32),
            pltpu.SemaphoreType.DMA,
        ],
        compiler_params=pltpu.CompilerParams(use_tc_tiling_on_sc=False),
    )
    def gather(users_hbm, movies_hbm, ut_hbm, mt_hbm, uout_hbm, mout_hbm,
               uidx, urows, midx, mrows, sem):
        wid = lax.axis_index("s") * NC + lax.axis_index("c")
        base = wid * B_PER_W
        for j in range(NCHUNK):
            pltpu.sync_copy(users_hbm.at[pl.ds(base + j * CHUNK, CHUNK)],
                            uidx.at[j])
            pltpu.sync_copy(movies_hbm.at[pl.ds(base + j * CHUNK, CHUNK)],
                            midx.at[j])
        copies = []
        for j in range(NCHUNK):
            copies.append(pltpu.async_copy(
                ut_hbm.at[uidx.at[j]], urows.at[pl.ds(j * CHUNK, CHUNK)], sem))
            copies.append(pltpu.async_copy(
                mt_hbm.at[midx.at[j]], mrows.at[pl.ds(j * CHUNK, CHUNK)], sem))
        for c in copies:
            c.wait()
        pltpu.sync_copy(urows, uout_hbm.at[pl.ds(base, B_PER_W)])
        pltpu.sync_copy(mrows, mout_hbm.at[pl.ds(base, B_PER_W)])

    return gather


MBLK = 2048


def _r16(x):
    return x.astype(jnp.bfloat16).astype(jnp.float32)


def _mlp_body(ue, me, w1a, w1b, b1, w2, b2, w3, b3, out):
    h = jnp.dot(_r16(ue[...]), w1a[...], preferred_element_type=jnp.float32,
                precision=lax.Precision.HIGHEST)
    h = h + jnp.dot(_r16(me[...]), w1b[...], preferred_element_type=jnp.float32,
                    precision=lax.Precision.HIGHEST)
    h = _r16(jnp.maximum(h + b1[...], 0.0))
    h = _r16(jnp.maximum(
        jnp.dot(h, w2[...], preferred_element_type=jnp.float32,
                precision=lax.Precision.HIGHEST) + b2[...], 0.0))
    out[...] = jnp.maximum(jnp.sum(h * w3[...], axis=1) + b3[0, 0], 0.0)


def kernel(users, movies, user_table, movie_table, W1, b1, W2, b2, W3, b3):
    ue, me = _build_gather()(users.astype(jnp.int32), movies.astype(jnp.int32),
                             user_table, movie_table)
    out = pl.pallas_call(
        _mlp_body,
        grid=(B // MBLK,),
        in_specs=[
            pl.BlockSpec((MBLK, D), lambda i: (i, 0)),
            pl.BlockSpec((MBLK, D), lambda i: (i, 0)),
            pl.BlockSpec((D, 64), lambda i: (0, 0)),
            pl.BlockSpec((D, 64), lambda i: (0, 0)),
            pl.BlockSpec((1, 64), lambda i: (0, 0)),
            pl.BlockSpec((64, 16), lambda i: (0, 0)),
            pl.BlockSpec((1, 16), lambda i: (0, 0)),
            pl.BlockSpec((1, 16), lambda i: (0, 0)),
            pl.BlockSpec((1, 1), lambda i: (0, 0)),
        ],
        out_specs=pl.BlockSpec((MBLK,), lambda i: (i,)),
        out_shape=jax.ShapeDtypeStruct((B,), jnp.float32),
    )(ue, me, W1[:D], W1[D:], b1.reshape(1, 64), W2, b2.reshape(1, 16),
      W3.reshape(1, 16), b3.reshape(1, 1))
    return out
